# baseline (speedup 1.0000x reference)
import functools

import jax
import jax.numpy as jnp
from jax import lax
from jax.experimental import pallas as pl
from jax.experimental.pallas import tpu as pltpu
from jax.experimental.pallas import tpu_sc as plsc

N_CAT = 26
CARD = 100000
N_NUM = 13
D_H = 16
D_M = 64

NC, NS = 2, 16
NW = NC * NS
LANES = 16

SEG = 100096
GCHUNK = 8192

J_SPLITS = ((0, 8), (8, 14), (14, 20), (20, 26))


def _sc_gather_body(batch, j0, nj, xcat_hbm, table_hbm, out_hbm,
                    xv, seg_v, gbuf, sem):
    wid = lax.axis_index("s") * NC + lax.axis_index("c")
    per_w = nj * D_H // NW
    for q in range(per_w):
        p = wid * per_w + q
        jl = p // D_H
        c = p % D_H
        j = j0 + jl
        lo = (j * CARD) // 128 * 128
        rel = j * CARD - lo
        pltpu.sync_copy(xcat_hbm.at[j], xv)
        pltpu.async_copy(table_hbm.at[c].at[pl.ds(lo, SEG)], seg_v, sem).wait()
        for h in range(batch // GCHUNK):

            def body(i, _, h=h):
                sl = pl.ds(h * GCHUNK + i * LANES, LANES)
                v = xv[sl] + rel
                gbuf[pl.ds(i * LANES, LANES)] = plsc.load_gather(seg_v, [v])
                return 0

            lax.fori_loop(0, GCHUNK // LANES, body, 0, unroll=8)
            pltpu.sync_copy(gbuf, out_hbm.at[jl, c, pl.ds(h * GCHUNK, GCHUNK)])


def _sc_gather(x_cat_t, emb_table_t, j0, nj):
    batch = x_cat_t.shape[1]
    mesh = plsc.VectorSubcoreMesh(core_axis_name="c", subcore_axis_name="s",
                                  num_cores=NC, num_subcores=NS)
    body = functools.partial(_sc_gather_body, batch, j0, nj)
    run = pl.kernel(
        body,
        out_type=jax.ShapeDtypeStruct((nj, D_H, batch), jnp.float32),
        mesh=mesh,
        scratch_types=[
            pltpu.VMEM((batch,), jnp.int32),
            pltpu.VMEM((SEG,), jnp.float32),
            pltpu.VMEM((GCHUNK,), jnp.float32),
            pltpu.SemaphoreType.DMA,
        ],
        compiler_params=pltpu.CompilerParams(needs_layout_passes=False,
                                             use_tc_tiling_on_sc=True,
                                             disable_bounds_checks=True),
        name=f"sc_gather_j{j0}",
    )
    return run(x_cat_t, emb_table_t)


def _tc_body(row0, with_num, xnum_ref, nw_ref, nbias_ref, nproj_ref, cbt_ref,
             cp_ref, gath_ref, *rest, has_prev):
    out_ref = rest[-1]
    bb = xnum_ref.shape[1]
    r = pl.program_id(0)
    cpT = jnp.transpose(cp_ref[:])
    bcT = jnp.dot(cpT, cbt_ref[:],
                  preferred_element_type=jnp.float32)

    def num_part():
        npT = jnp.transpose(nproj_ref[:])
        w2T = jnp.dot(npT, jnp.transpose(nw_ref[:]),
                      preferred_element_type=jnp.float32)
        b2T = jnp.dot(npT, jnp.transpose(nbias_ref[:]),
                      preferred_element_type=jnp.float32)
        k = r
        onehot = (lax.broadcasted_iota(jnp.int32, (N_NUM, 1), 0) == k
                  ).astype(jnp.float32)
        xk = jnp.dot(jnp.transpose(onehot), xnum_ref[:],
                     preferred_element_type=jnp.float32)
        wk = jnp.dot(w2T, onehot, preferred_element_type=jnp.float32)
        bk = jnp.dot(b2T, onehot, preferred_element_type=jnp.float32)
        return (lax.broadcast_in_dim(xk[0], (D_M, bb), (1,))
                * lax.broadcast_in_dim(wk[:, 0], (D_M, bb), (0,))
                + lax.broadcast_in_dim(bk[:, 0], (D_M, bb), (0,)))

    def cat_part():
        j = (r - N_NUM) if with_num else (row0 - N_NUM + r)
        gj = gath_ref[0]
        y = jnp.dot(cpT, gj, preferred_element_type=jnp.float32)
        onehot = (lax.broadcasted_iota(jnp.int32, (N_CAT, 1), 0) == j
                  ).astype(jnp.float32)
        bj = jnp.dot(bcT, onehot, preferred_element_type=jnp.float32)
        return y + lax.broadcast_in_dim(bj[:, 0], (D_M, bb), (0,))

    if with_num:
        val = lax.cond(r < N_NUM, num_part, cat_part)
    else:
        val = cat_part()
    out_ref[...] = val[None]


def _tc_chunk(x_num_t, gath, prev_out, weights, row0, nrows, with_num):
    num_weight, num_bias, num_proj, cat_bias_t, cat_proj = weights
    batch = x_num_t.shape[1]
    bb = 512
    nj = gath.shape[0]
    n_out = N_NUM + N_CAT
    grid = (nrows, batch // bb)

    def gmap(r, i, with_num=with_num):
        jl = (jnp.maximum(r - N_NUM, 0) if with_num else r)
        return (jl, 0, i)

    in_specs = [
        pl.BlockSpec((N_NUM, bb), lambda r, i: (0, i)),
        pl.BlockSpec((N_NUM, D_H), lambda r, i: (0, 0)),
        pl.BlockSpec((N_NUM, D_H), lambda r, i: (0, 0)),
        pl.BlockSpec((D_H, D_M), lambda r, i: (0, 0)),
        pl.BlockSpec((D_H, N_CAT), lambda r, i: (0, 0)),
        pl.BlockSpec((D_H, D_M), lambda r, i: (0, 0)),
        pl.BlockSpec((1, D_H, bb), gmap),
    ]
    args = [x_num_t, num_weight, num_bias, num_proj, cat_bias_t, cat_proj,
            gath]
    aliases = {}
    if prev_out is not None:
        in_specs.append(pl.BlockSpec(memory_space=pl.ANY))
        args.append(prev_out)
        aliases = {7: 0}
    body = functools.partial(_tc_body, row0, with_num,
                             has_prev=prev_out is not None)
    return pl.pallas_call(
        body,
        grid=grid,
        in_specs=in_specs,
        out_specs=pl.BlockSpec((1, D_M, bb), lambda r, i: (row0 + r, 0, i)),
        out_shape=jax.ShapeDtypeStruct((n_out, D_M, batch), jnp.float32),
        input_output_aliases=aliases,
        name=f"tc_rows{row0}",
    )(*args)


def kernel(x_num, x_cat, num_weight, num_bias, num_proj, emb_table, cat_bias,
           cat_proj):
    batch = x_cat.shape[0]
    x_cat_t = x_cat.T
    emb_table_t = emb_table.T
    x_num_t = x_num.T
    weights = (num_weight, num_bias, num_proj, cat_bias.T, cat_proj)
    n_out = N_NUM + N_CAT

    gaths = [_sc_gather(x_cat_t, emb_table_t, j0, j1 - j0)
             for (j0, j1) in J_SPLITS]

    del n_out, batch
    out = None
    for ci, (j0, j1) in enumerate(J_SPLITS):
        with_num = ci == 0
        row0 = 0 if with_num else N_NUM + j0
        nrows = (N_NUM if with_num else 0) + (j1 - j0)
        out = _tc_chunk(x_num_t, gaths[ci], out, weights, row0, nrows,
                        with_num)
    return jnp.transpose(out, (2, 0, 1))


# speedup vs baseline: 3.0985x; 3.0985x over previous
import functools

import jax
import jax.numpy as jnp
from jax import lax
from jax.experimental import pallas as pl
from jax.experimental.pallas import tpu as pltpu
from jax.experimental.pallas import tpu_sc as plsc

N_CAT = 26
CARD = 100000
N_NUM = 13
D_H = 16
D_M = 64

NC, NS = 2, 16
NW = NC * NS
LANES = 16
PLANES = N_CAT * D_H


SEG = 100096
GCHUNK = 8192


def _sc_gather_body(batch, xcat_hbm, table_hbm, out_hbm, xv, seg_v, gbuf, sem):
    wid = lax.axis_index("s") * NC + lax.axis_index("c")
    per_w = PLANES // NW
    iota = lax.iota(jnp.int32, LANES)
    del iota
    for q in range(per_w):
        p = wid * per_w + q
        j = p // D_H
        c = p % D_H
        lo = (j * CARD) // 128 * 128
        rel = j * CARD - lo
        pltpu.sync_copy(xcat_hbm.at[j], xv)
        pltpu.async_copy(table_hbm.at[c].at[pl.ds(lo, SEG)], seg_v, sem).wait()
        for h in range(batch // GCHUNK):

            def body(i, _, h=h):
                sl = pl.ds(h * GCHUNK + i * LANES, LANES)
                v = xv[sl] + rel
                gbuf[pl.ds(i * LANES, LANES)] = plsc.load_gather(seg_v, [v])
                return 0

            lax.fori_loop(0, GCHUNK // LANES, body, 0, unroll=8)
            pltpu.sync_copy(gbuf, out_hbm.at[j, c, pl.ds(h * GCHUNK, GCHUNK)])


def _sc_gather(x_cat_t, emb_table_t):
    batch = x_cat_t.shape[1]
    mesh = plsc.VectorSubcoreMesh(core_axis_name="c", subcore_axis_name="s",
                                  num_cores=NC, num_subcores=NS)
    body = functools.partial(_sc_gather_body, batch)
    run = pl.kernel(
        body,
        out_type=jax.ShapeDtypeStruct((N_CAT, D_H, batch), jnp.float32),
        mesh=mesh,
        scratch_types=[
            pltpu.VMEM((batch,), jnp.int32),
            pltpu.VMEM((SEG,), jnp.float32),
            pltpu.VMEM((GCHUNK,), jnp.float32),
            pltpu.SemaphoreType.DMA,
        ],
        compiler_params=pltpu.CompilerParams(needs_layout_passes=False,
                                             use_tc_tiling_on_sc=True,
                                             disable_bounds_checks=True),
    )
    return run(x_cat_t, emb_table_t)


def _tc_body(xnum_ref, nw_ref, nbias_ref, nproj_ref, cbt_ref, cp_ref,
             gath_ref, out_ref):
    bb = xnum_ref.shape[1]
    npT = jnp.transpose(nproj_ref[:])
    w2T = jnp.dot(npT, jnp.transpose(nw_ref[:]),
                  preferred_element_type=jnp.float32)
    b2T = jnp.dot(npT, jnp.transpose(nbias_ref[:]),
                  preferred_element_type=jnp.float32)
    cpT = jnp.transpose(cp_ref[:])
    bcT = jnp.dot(cpT, cbt_ref[:],
                  preferred_element_type=jnp.float32)

    for k in range(N_NUM):
        xk = lax.broadcast_in_dim(xnum_ref[k], (D_M, bb), (1,))
        wk = lax.broadcast_in_dim(w2T[:, k], (D_M, bb), (0,))
        bk = lax.broadcast_in_dim(b2T[:, k], (D_M, bb), (0,))
        out_ref[pl.ds(k, 1)] = (xk * wk + bk)[None]

    for j in range(N_CAT):
        gj = gath_ref[j]
        y = jnp.dot(cpT, gj, preferred_element_type=jnp.float32)
        bj = lax.broadcast_in_dim(bcT[:, j], (D_M, bb), (0,))
        out_ref[pl.ds(N_NUM + j, 1)] = (y + bj)[None]


def _tc_fused(x_num_t, gath, num_weight, num_bias, num_proj, cat_bias_t,
              cat_proj):
    batch = x_num_t.shape[1]
    bb = 512
    grid = (batch // bb,)
    n_out = N_NUM + N_CAT
    return pl.pallas_call(
        _tc_body,
        grid=grid,
        in_specs=[
            pl.BlockSpec((N_NUM, bb), lambda i: (0, i)),
            pl.BlockSpec((N_NUM, D_H), lambda i: (0, 0)),
            pl.BlockSpec((N_NUM, D_H), lambda i: (0, 0)),
            pl.BlockSpec((D_H, D_M), lambda i: (0, 0)),
            pl.BlockSpec((D_H, N_CAT), lambda i: (0, 0)),
            pl.BlockSpec((D_H, D_M), lambda i: (0, 0)),
            pl.BlockSpec((N_CAT, D_H, bb), lambda i: (0, 0, i)),
        ],
        out_specs=pl.BlockSpec((n_out, D_M, bb), lambda i: (0, 0, i)),
        out_shape=jax.ShapeDtypeStruct((n_out, D_M, batch), jnp.float32),
    )(x_num_t, num_weight, num_bias, num_proj, cat_bias_t, cat_proj, gath)


def kernel(x_num, x_cat, num_weight, num_bias, num_proj, emb_table, cat_bias,
           cat_proj):
    gath = _sc_gather(x_cat.T, emb_table.T)
    out_t = _tc_fused(x_num.T, gath, num_weight, num_bias, num_proj,
                      cat_bias.T, cat_proj)
    return jnp.transpose(out_t, (2, 0, 1))


# speedup vs baseline: 3.3225x; 1.0723x over previous
import functools

import jax
import jax.numpy as jnp
from jax import lax
from jax.experimental import pallas as pl
from jax.experimental.pallas import tpu as pltpu
from jax.experimental.pallas import tpu_sc as plsc

N_CAT = 26
CARD = 100000
N_NUM = 13
D_H = 16
D_M = 64

NC, NS = 2, 16
NW = NC * NS
LANES = 16
PLANES = N_CAT * D_H


SEG = 100096
GCHUNK = 4096


def _sc_gather_body(batch, xcat_hbm, table_hbm, out_hbm, xv, seg_v, gb0, gb1,
                    sem, wsem):
    wid = lax.axis_index("s") * NC + lax.axis_index("c")
    per_w = PLANES // NW
    gbufs = (gb0, gb1)
    nh = batch // GCHUNK
    for q in range(per_w):
        p = wid * per_w + q
        j = p // D_H
        c = p % D_H
        lo = (j * CARD) // 128 * 128
        rel = j * CARD - lo
        if q == 0:
            pltpu.sync_copy(xcat_hbm.at[j], xv)
        else:
            jprev = (p - 1) // D_H

            @pl.when(j != jprev)
            def _():
                pltpu.sync_copy(xcat_hbm.at[j], xv)

        pltpu.async_copy(table_hbm.at[c].at[pl.ds(lo, SEG)], seg_v, sem).wait()
        writes = []
        for h in range(nh):
            gbuf = gbufs[h % 2]
            if len(writes) >= 2:
                writes.pop(0).wait()

            def body(i, _, h=h, gbuf=gbuf):
                sl = pl.ds(h * GCHUNK + i * LANES, LANES)
                v = xv[sl] + rel
                gbuf[pl.ds(i * LANES, LANES)] = plsc.load_gather(seg_v, [v])
                return 0

            lax.fori_loop(0, GCHUNK // LANES, body, 0, unroll=8)
            writes.append(
                pltpu.async_copy(gbuf,
                                 out_hbm.at[j, c, pl.ds(h * GCHUNK, GCHUNK)],
                                 wsem))
        for w in writes:
            w.wait()


def _sc_gather(x_cat_t, emb_table_t):
    batch = x_cat_t.shape[1]
    mesh = plsc.VectorSubcoreMesh(core_axis_name="c", subcore_axis_name="s",
                                  num_cores=NC, num_subcores=NS)
    body = functools.partial(_sc_gather_body, batch)
    run = pl.kernel(
        body,
        out_type=jax.ShapeDtypeStruct((N_CAT, D_H, batch), jnp.float32),
        mesh=mesh,
        scratch_types=[
            pltpu.VMEM((batch,), jnp.int32),
            pltpu.VMEM((SEG,), jnp.float32),
            pltpu.VMEM((GCHUNK,), jnp.float32),
            pltpu.VMEM((GCHUNK,), jnp.float32),
            pltpu.SemaphoreType.DMA,
            pltpu.SemaphoreType.DMA,
        ],
        compiler_params=pltpu.CompilerParams(needs_layout_passes=False,
                                             use_tc_tiling_on_sc=True,
                                             disable_bounds_checks=True),
    )
    return run(x_cat_t, emb_table_t)


def _tc_body(xnum_ref, nw_ref, nbias_ref, nproj_ref, cbt_ref, cp_ref,
             gath_ref, out_ref):
    bb = xnum_ref.shape[1]
    npT = jnp.transpose(nproj_ref[:])
    w2T = jnp.dot(npT, jnp.transpose(nw_ref[:]),
                  preferred_element_type=jnp.float32)
    b2T = jnp.dot(npT, jnp.transpose(nbias_ref[:]),
                  preferred_element_type=jnp.float32)
    cpT = jnp.transpose(cp_ref[:])
    bcT = jnp.dot(cpT, cbt_ref[:],
                  preferred_element_type=jnp.float32)

    for k in range(N_NUM):
        xk = lax.broadcast_in_dim(xnum_ref[k], (D_M, bb), (1,))
        wk = lax.broadcast_in_dim(w2T[:, k], (D_M, bb), (0,))
        bk = lax.broadcast_in_dim(b2T[:, k], (D_M, bb), (0,))
        out_ref[pl.ds(k, 1)] = (xk * wk + bk)[None]

    for j in range(N_CAT):
        gj = gath_ref[j]
        y = jnp.dot(cpT, gj, preferred_element_type=jnp.float32)
        bj = lax.broadcast_in_dim(bcT[:, j], (D_M, bb), (0,))
        out_ref[pl.ds(N_NUM + j, 1)] = (y + bj)[None]


def _tc_fused(x_num_t, gath, num_weight, num_bias, num_proj, cat_bias_t,
              cat_proj):
    batch = x_num_t.shape[1]
    bb = 512
    grid = (batch // bb,)
    n_out = N_NUM + N_CAT
    return pl.pallas_call(
        _tc_body,
        grid=grid,
        in_specs=[
            pl.BlockSpec((N_NUM, bb), lambda i: (0, i)),
            pl.BlockSpec((N_NUM, D_H), lambda i: (0, 0)),
            pl.BlockSpec((N_NUM, D_H), lambda i: (0, 0)),
            pl.BlockSpec((D_H, D_M), lambda i: (0, 0)),
            pl.BlockSpec((D_H, N_CAT), lambda i: (0, 0)),
            pl.BlockSpec((D_H, D_M), lambda i: (0, 0)),
            pl.BlockSpec((N_CAT, D_H, bb), lambda i: (0, 0, i)),
        ],
        out_specs=pl.BlockSpec((n_out, D_M, bb), lambda i: (0, 0, i)),
        out_shape=jax.ShapeDtypeStruct((n_out, D_M, batch), jnp.float32),
    )(x_num_t, num_weight, num_bias, num_proj, cat_bias_t, cat_proj, gath)


def kernel(x_num, x_cat, num_weight, num_bias, num_proj, emb_table, cat_bias,
           cat_proj):
    gath = _sc_gather(x_cat.T, emb_table.T)
    out_t = _tc_fused(x_num.T, gath, num_weight, num_bias, num_proj,
                      cat_bias.T, cat_proj)
    return jnp.transpose(out_t, (2, 0, 1))


# speedup vs baseline: 3.3474x; 1.0075x over previous
import functools

import jax
import jax.numpy as jnp
from jax import lax
from jax.experimental import pallas as pl
from jax.experimental.pallas import tpu as pltpu
from jax.experimental.pallas import tpu_sc as plsc

N_CAT = 26
CARD = 100000
N_NUM = 13
D_H = 16
D_M = 64

NC, NS = 2, 16
NW = NC * NS
LANES = 16
PLANES = N_CAT * D_H


SEG = 100096
GCHUNK = 4096


def _sc_gather_body(batch, xcat_hbm, table_hbm, out_hbm, xv, seg_v, gb0, gb1,
                    sem, wsem):
    wid = lax.axis_index("s") * NC + lax.axis_index("c")
    per_w = PLANES // NW
    gbufs = (gb0, gb1)
    nh = batch // GCHUNK
    for q in range(per_w):
        p = wid * per_w + q
        j = p // D_H
        c = p % D_H
        lo = (j * CARD) // 128 * 128
        rel = j * CARD - lo
        if q == 0:
            pltpu.sync_copy(xcat_hbm.at[j], xv)
        else:
            jprev = (p - 1) // D_H

            @pl.when(j != jprev)
            def _():
                pltpu.sync_copy(xcat_hbm.at[j], xv)

        pltpu.async_copy(table_hbm.at[c].at[pl.ds(lo, SEG)], seg_v, sem).wait()
        writes = []
        for h in range(nh):
            gbuf = gbufs[h % 2]
            if len(writes) >= 2:
                writes.pop(0).wait()

            def body(i, _, h=h, gbuf=gbuf):
                sl = pl.ds(h * GCHUNK + i * LANES, LANES)
                v = xv[sl] + rel
                gbuf[pl.ds(i * LANES, LANES)] = plsc.load_gather(seg_v, [v])
                return 0

            lax.fori_loop(0, GCHUNK // LANES, body, 0, unroll=16)
            writes.append(
                pltpu.async_copy(gbuf,
                                 out_hbm.at[j, c, pl.ds(h * GCHUNK, GCHUNK)],
                                 wsem))
        for w in writes:
            w.wait()


def _sc_gather(x_cat_t, emb_table_t):
    batch = x_cat_t.shape[1]
    mesh = plsc.VectorSubcoreMesh(core_axis_name="c", subcore_axis_name="s",
                                  num_cores=NC, num_subcores=NS)
    body = functools.partial(_sc_gather_body, batch)
    run = pl.kernel(
        body,
        out_type=jax.ShapeDtypeStruct((N_CAT, D_H, batch), jnp.float32),
        mesh=mesh,
        scratch_types=[
            pltpu.VMEM((batch,), jnp.int32),
            pltpu.VMEM((SEG,), jnp.float32),
            pltpu.VMEM((GCHUNK,), jnp.float32),
            pltpu.VMEM((GCHUNK,), jnp.float32),
            pltpu.SemaphoreType.DMA,
            pltpu.SemaphoreType.DMA,
        ],
        compiler_params=pltpu.CompilerParams(needs_layout_passes=False,
                                             use_tc_tiling_on_sc=True,
                                             disable_bounds_checks=True),
    )
    return run(x_cat_t, emb_table_t)


def _tc_body(xnum_ref, nw_ref, nbias_ref, nproj_ref, cbt_ref, cp_ref,
             gath_ref, out_ref):
    bb = xnum_ref.shape[1]
    npT = jnp.transpose(nproj_ref[:])
    w2T = jnp.dot(npT, jnp.transpose(nw_ref[:]),
                  preferred_element_type=jnp.float32)
    b2T = jnp.dot(npT, jnp.transpose(nbias_ref[:]),
                  preferred_element_type=jnp.float32)
    cpT = jnp.transpose(cp_ref[:])
    bcT = jnp.dot(cpT, cbt_ref[:],
                  preferred_element_type=jnp.float32)

    for k in range(N_NUM):
        xk = lax.broadcast_in_dim(xnum_ref[k], (D_M, bb), (1,))
        wk = lax.broadcast_in_dim(w2T[:, k], (D_M, bb), (0,))
        bk = lax.broadcast_in_dim(b2T[:, k], (D_M, bb), (0,))
        out_ref[pl.ds(k, 1)] = (xk * wk + bk)[None]

    for j in range(N_CAT):
        gj = gath_ref[j]
        y = jnp.dot(cpT, gj, preferred_element_type=jnp.float32)
        bj = lax.broadcast_in_dim(bcT[:, j], (D_M, bb), (0,))
        out_ref[pl.ds(N_NUM + j, 1)] = (y + bj)[None]


def _tc_fused(x_num_t, gath, num_weight, num_bias, num_proj, cat_bias_t,
              cat_proj):
    batch = x_num_t.shape[1]
    bb = 1024
    grid = (batch // bb,)
    n_out = N_NUM + N_CAT
    return pl.pallas_call(
        _tc_body,
        grid=grid,
        in_specs=[
            pl.BlockSpec((N_NUM, bb), lambda i: (0, i)),
            pl.BlockSpec((N_NUM, D_H), lambda i: (0, 0)),
            pl.BlockSpec((N_NUM, D_H), lambda i: (0, 0)),
            pl.BlockSpec((D_H, D_M), lambda i: (0, 0)),
            pl.BlockSpec((D_H, N_CAT), lambda i: (0, 0)),
            pl.BlockSpec((D_H, D_M), lambda i: (0, 0)),
            pl.BlockSpec((N_CAT, D_H, bb), lambda i: (0, 0, i)),
        ],
        out_specs=pl.BlockSpec((n_out, D_M, bb), lambda i: (0, 0, i)),
        out_shape=jax.ShapeDtypeStruct((n_out, D_M, batch), jnp.float32),
    )(x_num_t, num_weight, num_bias, num_proj, cat_bias_t, cat_proj, gath)


def kernel(x_num, x_cat, num_weight, num_bias, num_proj, emb_table, cat_bias,
           cat_proj):
    gath = _sc_gather(x_cat.T, emb_table.T)
    out_t = _tc_fused(x_num.T, gath, num_weight, num_bias, num_proj,
                      cat_bias.T, cat_proj)
    return jnp.transpose(out_t, (2, 0, 1))
